# baseline (speedup 1.0000x reference)
import functools

import jax
import jax.numpy as jnp
from jax import lax
from jax.experimental import pallas as pl
from jax.experimental.pallas import tpu as pltpu
from jax.experimental.pallas import tpu_sc as plsc

_VOCAB = 1000000
_D = 16
_B = 16384
_EPS = 1e-8
_TW = 40960
_TN = -(-_VOCAB // _TW)
_BR = _B // 8
_OUTROWS = _B + 128
_ORW = _OUTROWS * _D // 128
_W = 640
_NWING = -(-_VOCAB // _W)
_TAILW = _VOCAB - (_NWING - 1) * _W
_KMAX = -(-_NWING // 32)
_CAP = 8192
_RING = 16


@functools.cache
def _make_sc_gather():
    info = plsc.get_sparse_core_info()
    nc, ns = info.num_cores, info.num_subcores
    mesh = plsc.VectorSubcoreMesh(core_axis_name="c", subcore_axis_name="s")
    out = jax.ShapeDtypeStruct((_OUTROWS * _D,), jnp.float32)

    @functools.partial(
        pl.kernel,
        mesh=mesh,
        out_type=(out, out, out, out),
        scratch_types=[
            pltpu.VMEM((_B,), jnp.int32),
            pltpu.VMEM((_B,), jnp.int32),
            pltpu.VMEM((_CAP + 16,), jnp.int32),
            pltpu.VMEM((_CAP + 16,), jnp.int32),
            pltpu.VMEM((_CAP + 16,), jnp.int32),
            pltpu.VMEM((2, _D, _W), jnp.float32),
            pltpu.VMEM((2, _D, _W), jnp.float32),
            pltpu.VMEM((_RING, 256), jnp.float32),
            pltpu.VMEM((_RING, 256), jnp.float32),
            pltpu.VMEM((_RING, 2, 128), jnp.int32),
            pltpu.SemaphoreType.DMA((2,)),
            pltpu.SemaphoreType.DMA((_RING,)),
        ],
        compiler_params=pltpu.CompilerParams(
            use_tc_tiling_on_sc=True, needs_layout_passes=False),
    )
    def gather(min_hbm, del_hbm, i1_hbm, i2_hbm,
               o1m, o1d, o2m, o2d,
               idx1, idx2, pos1, pos2, wl,
               winm, wind, stm, std, sidx, wsem, ssem):
        wid = lax.axis_index("s") * nc + lax.axis_index("c")
        iota = lax.broadcasted_iota(jnp.int32, (16,), 0)

        pltpu.sync_copy(i1_hbm, idx1)
        pltpu.sync_copy(i2_hbm, idx2)

        def filt(idx_v, pos_v):
            def fb(k, cnt):
                v = idx_v[pl.ds(k * 16, 16)]
                m = lax.rem(lax.div(v, _W), 32) == wid
                plsc.store_compressed(pos_v.at[pl.ds(cnt, 16)],
                                      iota + k * 16, mask=m)
                return jnp.minimum(cnt + jnp.sum(m.astype(jnp.int32)), _CAP)
            return lax.fori_loop(0, _B // 16, fb, 0)

        cnt1 = filt(idx1, pos1)
        cnt2 = filt(idx2, pos2)

        def woff(w):
            return pl.multiple_of(
                jnp.where(w == _NWING - 1, (_NWING - 1) * _W - 384, w * _W),
                128)

        def fire_window(w, slot):
            w0 = woff(w)
            pltpu.async_copy(min_hbm.at[:, pl.ds(w0, _W)],
                             winm.at[slot], wsem.at[slot])
            pltpu.async_copy(del_hbm.at[:, pl.ds(w0, _W)],
                             wind.at[slot], wsem.at[slot])

        def wait_window(w, slot):
            w0 = woff(w)
            pltpu.make_async_copy(min_hbm.at[:, pl.ds(w0, _W)],
                                  winm.at[slot], wsem.at[slot]).wait()
            pltpu.make_async_copy(del_hbm.at[:, pl.ds(w0, _W)],
                                  wind.at[slot], wsem.at[slot]).wait()

        fire_window(wid, 0)

        def wloop(k, gcnt):
            w = wid + 32 * k
            slot = lax.rem(k, 2)

            @pl.when(wid + 32 * (k + 1) < _NWING)
            def _():
                fire_window(wid + 32 * (k + 1), lax.rem(k + 1, 2))

            def process(gcnt):
                wait_window(w, slot)
                w0 = woff(w)

                def do_set(idx_v, pos_v, cnt, om, od, gc0):
                    def cb(g, cw):
                        lane_ok = (iota + g * 16) < cnt
                        p16 = jnp.clip(pos_v[pl.ds(g * 16, 16)], 0, _B - 1)
                        vv = plsc.load_gather(idx_v, [p16], mask=lane_ok)
                        m = (lax.div(vv, _W) == w) & lane_ok
                        plsc.store_compressed(wl.at[pl.ds(cw, 16)], p16,
                                              mask=m)
                        return cw + jnp.sum(m.astype(jnp.int32))

                    cw = lax.fori_loop(0, (cnt + 15) // 16, cb, 0)

                    def eb(h, gc):
                        r = lax.rem(gc, _RING)

                        @pl.when(gc >= _RING)
                        def _():
                            for half in range(2):
                                pltpu.make_async_copy(
                                    stm.at[r, pl.ds(half * 128, 128)],
                                    om.at[sidx.at[r, half]],
                                    ssem.at[r]).wait()
                                pltpu.make_async_copy(
                                    std.at[r, pl.ds(half * 128, 128)],
                                    od.at[sidx.at[r, half]],
                                    ssem.at[r]).wait()

                        valid = (iota + h * 16) < cw
                        wp16 = jnp.clip(wl[pl.ds(h * 16, 16)], 0, _B - 1)
                        vv = plsc.load_gather(idx_v, [wp16], mask=valid)
                        vloc = jnp.clip(vv - w0, 0, _W - 1)
                        pos = jnp.where(valid, wp16, _B + iota)
                        for b in range(16):
                            vb = jnp.full((16,), vloc[b], jnp.int32)
                            stm[r, pl.ds(b * 16, 16)] = plsc.load_gather(
                                winm.at[slot], [iota, vb])
                            std[r, pl.ds(b * 16, 16)] = plsc.load_gather(
                                wind.at[slot], [iota, vb])
                            sidx[r, b // 8, pl.ds((b % 8) * 16, 16)] = (
                                jnp.full((16,), pos[b] * _D, jnp.int32) + iota)
                        for half in range(2):
                            pltpu.async_copy(stm.at[r, pl.ds(half * 128, 128)],
                                             om.at[sidx.at[r, half]],
                                             ssem.at[r])
                            pltpu.async_copy(std.at[r, pl.ds(half * 128, 128)],
                                             od.at[sidx.at[r, half]],
                                             ssem.at[r])
                        return gc + 1

                    return lax.fori_loop(0, (cw + 15) // 16, eb, gc0)

                gcnt = do_set(idx1, pos1, cnt1, o1m, o1d, gcnt)
                gcnt = do_set(idx2, pos2, cnt2, o2m, o2d, gcnt)
                return gcnt

            return lax.cond(w < _NWING, process, lambda g: g, gcnt)

        gcnt = lax.fori_loop(0, _KMAX, wloop, 0)

        def drain(k, _):
            r = lax.rem(k, _RING)

            @pl.when(k < jnp.minimum(gcnt, _RING))
            def _():
                for half in range(2):
                    pltpu.make_async_copy(
                        stm.at[r, pl.ds(half * 128, 128)],
                        o1m.at[sidx.at[r, half]], ssem.at[r]).wait()
                    pltpu.make_async_copy(
                        std.at[r, pl.ds(half * 128, 128)],
                        o1d.at[sidx.at[r, half]], ssem.at[r]).wait()
            return 0

        lax.fori_loop(0, _RING, drain, 0)

    return gather


def _seg_sum(x, sel):
    return lax.dot_general(x, sel, (((1,), (0,)), ((), ())),
                           precision=lax.Precision.HIGHEST,
                           preferred_element_type=jnp.float32)


def _loss_body(t1m, t1d, t2m, t2d, lab, minb, delb, out_ref):
    i = pl.program_id(0)
    vcol = lax.broadcasted_iota(jnp.int32, (_D, _TW), 1) + i * _TW
    regv = jnp.where(vcol < _VOCAB, jnp.abs(1.0 - minb[...] - delb[...]), 0.0)
    reg = jnp.sum(regv)

    @pl.when(i == 0)
    def _():
        lane_grp = lax.broadcasted_iota(jnp.int32, (128, 8), 0) // _D
        grp = lax.broadcasted_iota(jnp.int32, (128, 8), 1)
        sel = (lane_grp == grp).astype(jnp.float32)

        a_lo = t1m[...]
        a_hi = a_lo + t1d[...]
        b_lo = t2m[...]
        b_hi = b_lo + t2d[...]
        meet_lo = jnp.maximum(a_lo, b_lo)
        meet_hi = jnp.minimum(a_hi, b_hi)
        join_lo = jnp.minimum(a_lo, b_lo)
        join_hi = jnp.maximum(a_hi, b_hi)

        def lv(lo, hi):
            return _seg_sum(jnp.log(jnp.clip(hi - lo, _EPS, None)), sel)

        log_meet = lv(meet_lo, meet_hi)
        log_join = lv(join_lo, join_hi)
        log_t1 = lv(a_lo, a_hi)
        log_t2 = lv(b_lo, b_hi)
        disj_cnt = _seg_sum((meet_hi <= meet_lo).astype(jnp.float32), sel)
        disj = disj_cnt > 0.0

        cond = log_meet - log_t2
        pos_overlap = -cond
        upper = jnp.clip(jnp.exp(log_join) - jnp.exp(log_t1) - jnp.exp(log_t2),
                         _EPS, None)
        pos_disjoint = -(jnp.log(upper) - log_t2)
        train_pos = jnp.where(disj, pos_disjoint, pos_overlap)
        neg_overlap = -jnp.log(jnp.clip(1.0 - jnp.exp(cond), _EPS, None))
        train_neg = jnp.where(disj, 0.0, neg_overlap)
        lb = lab[...]
        cond_loss = (jnp.sum(train_pos * lb)
                     + jnp.sum(train_neg * (1.0 - lb))) / (_B / 2)
        out_ref[0, 0] = cond_loss

    out_ref[0, 0] += reg * (0.0001 / _VOCAB)


def kernel(t1x, t2x, label, min_embed, delta_embed):
    i1 = t1x[:, 0].astype(jnp.int32)
    i2 = t2x[:, 0].astype(jnp.int32)
    min_t = jnp.swapaxes(min_embed, 0, 1)
    del_t = jnp.swapaxes(delta_embed, 0, 1)
    o1m, o1d, o2m, o2d = _make_sc_gather()(min_t, del_t, i1, i2)
    wide = lambda a: a.reshape(_ORW, 128)
    lab = label.reshape(_BR, 8)

    batch_spec = pl.BlockSpec((_BR, 128), lambda i: (0, 0))
    lab_spec = pl.BlockSpec((_BR, 8), lambda i: (0, 0))
    tab_spec = pl.BlockSpec((_D, _TW), lambda i: (0, i))
    loss = pl.pallas_call(
        _loss_body,
        grid=(_TN,),
        in_specs=[batch_spec, batch_spec, batch_spec, batch_spec,
                  lab_spec, tab_spec, tab_spec],
        out_specs=pl.BlockSpec(memory_space=pltpu.SMEM),
        out_shape=jax.ShapeDtypeStruct((1, 1), jnp.float32),
    )(wide(o1m), wide(o1d), wide(o2m), wide(o2d), lab, min_t, del_t)
    return loss[0, 0]
